# baseline (speedup 1.0000x reference)
import functools

import jax
import jax.numpy as jnp
from jax import lax
from jax.experimental import pallas as pl
from jax.experimental.pallas import tpu as pltpu
from jax.experimental.pallas import tpu_sc as plsc

R = 8
NVOX = R * R * R
LANES = 16
NC, NS = 2, 16
NTILES = NC * NS

CHUNK = 8192


def _hist_body(x_hbm, counts_hbm, buf, hist, cnt):
    B, N3 = x_hbm.shape
    N = N3 // 3
    bid = lax.axis_index("s") * NC + lax.axis_index("c")

    lane = lax.iota(jnp.int32, LANES)
    lane3_0 = lane * 3
    lane3_1 = lane * 3 + 1
    lane3_2 = lane * 3 + 2
    zeros16 = jnp.zeros((LANES,), jnp.float32)
    ones16 = jnp.ones((LANES,), jnp.float32)
    n_chunks = N // CHUNK
    n_groups = CHUNK // LANES

    def zero_body(i, _):
        hist[pl.ds(i * LANES, LANES)] = zeros16
        return 0

    lax.fori_loop(0, (LANES * NVOX) // LANES, zero_body, 0)

    def p1_chunk(c, carry):
        pltpu.sync_copy(x_hbm.at[bid, pl.ds(c * CHUNK * 3, CHUNK * 3)], buf)

        def p1_group(g, carry):
            mn0, mn1, mn2, mx0, mx1, mx2 = carry
            base = g * (LANES * 3)
            v0 = plsc.load_gather(buf, [base + lane3_0])
            v1 = plsc.load_gather(buf, [base + lane3_1])
            v2 = plsc.load_gather(buf, [base + lane3_2])
            return (jnp.minimum(mn0, v0), jnp.minimum(mn1, v1),
                    jnp.minimum(mn2, v2), jnp.maximum(mx0, v0),
                    jnp.maximum(mx1, v1), jnp.maximum(mx2, v2))

        return lax.fori_loop(0, n_groups, p1_group, carry)

    big = jnp.full((LANES,), jnp.inf, jnp.float32)
    carry = (big, big, big, -big, -big, -big)
    mn0, mn1, mn2, mx0, mx1, mx2 = lax.fori_loop(0, n_chunks, p1_chunk, carry)

    def splat(s):
        return jnp.broadcast_to(s, (LANES,))

    mn_0, mn_1, mn_2 = splat(jnp.min(mn0)), splat(jnp.min(mn1)), splat(jnp.min(mn2))
    mx_0, mx_1, mx_2 = splat(jnp.max(mx0)), splat(jnp.max(mx1)), splat(jnp.max(mx2))

    def scale_of(mn, mx):
        width = jnp.where(mx > mn, mx - mn, jnp.full((LANES,), 1.0, jnp.float32))
        return jnp.full((LANES,), float(R), jnp.float32) / width

    s0, s1, s2 = scale_of(mn_0, mx_0), scale_of(mn_1, mx_1), scale_of(mn_2, mx_2)

    def p2_chunk(c, _):
        pltpu.sync_copy(x_hbm.at[bid, pl.ds(c * CHUNK * 3, CHUNK * 3)], buf)

        def p2_group(g, _):
            base = g * (LANES * 3)
            v0 = plsc.load_gather(buf, [base + lane3_0])
            v1 = plsc.load_gather(buf, [base + lane3_1])
            v2 = plsc.load_gather(buf, [base + lane3_2])
            i0 = ((v0 - mn_0) * s0).astype(jnp.int32)
            i1 = ((v1 - mn_1) * s1).astype(jnp.int32)
            i2 = ((v2 - mn_2) * s2).astype(jnp.int32)
            i0 = jnp.minimum(jnp.maximum(i0, 0), R - 1)
            i1 = jnp.minimum(jnp.maximum(i1, 0), R - 1)
            i2 = jnp.minimum(jnp.maximum(i2, 0), R - 1)
            vox = (i0 * R + i1) * R + i2 + lane * NVOX
            plsc.addupdate_scatter(hist, [vox], ones16)
            return 0

        return lax.fori_loop(0, n_groups, p2_group, 0)

    lax.fori_loop(0, n_chunks, p2_chunk, 0)

    inv_n = jnp.float32(1.0 / N)

    def merge_body(g, _):
        acc = zeros16
        for l in range(LANES):
            acc = acc + hist[pl.ds(l * NVOX + g * LANES, LANES)]
        cnt[pl.ds(g * LANES, LANES)] = acc * inv_n
        return 0

    lax.fori_loop(0, NVOX // LANES, merge_body, 0)
    pltpu.sync_copy(cnt, counts_hbm.at[bid])


def _sc_counts(x):
    B, N, _ = x.shape
    mesh = plsc.VectorSubcoreMesh(core_axis_name="c", subcore_axis_name="s",
                                  num_cores=NC, num_subcores=NS)
    return pl.kernel(
        _hist_body,
        out_type=jax.ShapeDtypeStruct((B, NVOX), jnp.float32),
        mesh=mesh,
        compiler_params=pltpu.CompilerParams(
            needs_layout_passes=False, use_tc_tiling_on_sc=False),
        scratch_types=[
            pltpu.VMEM((CHUNK * 3,), jnp.float32),
            pltpu.VMEM((LANES * NVOX,), jnp.float32),
            pltpu.VMEM((NVOX,), jnp.float32),
        ],
    )(x.reshape(B, N * 3))


def _gemm_body(c_ref, w_ref, b_ref, o_ref):
    o_ref[...] = lax.dot_general(
        c_ref[...], w_ref[...], (((1,), (1,)), ((), ())),
        preferred_element_type=jnp.float32) + b_ref[...]


def _tc_gemm(counts, W, b):
    B = counts.shape[0]
    C = W.shape[0]
    return pl.pallas_call(
        _gemm_body,
        out_shape=jax.ShapeDtypeStruct((B, C), jnp.float32),
    )(counts, W, b.reshape(1, C))


@jax.jit
def kernel(x, W, b):
    counts = _sc_counts(x)
    return _tc_gemm(counts, W, b)


# speedup vs baseline: 3.4814x; 3.4814x over previous
import functools

import jax
import jax.numpy as jnp
from jax import lax
from jax.experimental import pallas as pl
from jax.experimental.pallas import tpu as pltpu
from jax.experimental.pallas import tpu_sc as plsc

R = 8
NVOX = R * R * R
LANES = 16
NC, NS = 2, 16

CHUNK = 16384
U1 = 16
U2 = 8


def _hist_body(x_hbm, counts_hbm, b0, b1, b2, b3, b4, b5, hist, cnt,
               s0_, s1_, s2_, s3_, s4_, s5_):
    _, B, N = x_hbm.shape
    bid = lax.axis_index("s") * NC + lax.axis_index("c")

    zeros16 = jnp.zeros((LANES,), jnp.float32)
    ones16 = jnp.ones((LANES,), jnp.float32)
    lane_off = lax.iota(jnp.int32, LANES) * NVOX
    n_chunks = N // CHUNK
    bufs = (b0, b1, b2, b3, b4, b5)
    sems = (s0_, s1_, s2_, s3_, s4_, s5_)

    def start(d, c, slot):
        return pltpu.async_copy(
            x_hbm.at[d, bid, pl.ds(c * CHUNK, CHUNK)], bufs[slot], sems[slot])

    pending = start(0, 0, 0)

    def zero_body(i, _):
        hist[pl.ds(i * LANES, LANES)] = zeros16
        return 0

    lax.fori_loop(0, (LANES * NVOX) // LANES, zero_body, 0)

    big = jnp.full((LANES,), jnp.inf, jnp.float32)
    n1_iters = CHUNK // (LANES * U1)
    minmax = []
    for d in range(3):
        carry = (big, -big)
        for c in range(n_chunks):
            slot = (d * n_chunks + c) % 2
            nslot = 1 - slot
            if c + 1 < n_chunks:
                nxt = start(d, c + 1, nslot)
            elif d + 1 < 3:
                nxt = start(d + 1, 0, nslot)
            else:
                nxt = None
            pending.wait()
            buf = bufs[slot]

            def p1_iter(it, carry, buf=buf):
                mn, mx = carry
                for k in range(U1):
                    v = buf[pl.ds((it * U1 + k) * LANES, LANES)]
                    mn = jnp.minimum(mn, v)
                    mx = jnp.maximum(mx, v)
                return (mn, mx)

            carry = lax.fori_loop(0, n1_iters, p1_iter, carry)
            pending = nxt
        minmax.append(carry)

    def splat(s):
        return jnp.broadcast_to(s, (LANES,))

    mns, scs = [], []
    for mn, mx in minmax:
        mn_s, mx_s = splat(jnp.min(mn)), splat(jnp.max(mx))
        width = jnp.where(mx_s > mn_s, mx_s - mn_s,
                          jnp.full((LANES,), 1.0, jnp.float32))
        mns.append(mn_s)
        scs.append(jnp.full((LANES,), float(R), jnp.float32) / width)
    mn_0, mn_1, mn_2 = mns
    sc_0, sc_1, sc_2 = scs

    n2_iters = CHUNK // (LANES * U2)

    def start3(c, phase):
        return [start(d, c, 3 * phase + d) for d in range(3)]

    pending3 = start3(0, 0)
    for c in range(n_chunks):
        phase = c % 2
        nxt3 = start3(c + 1, 1 - phase) if c + 1 < n_chunks else None
        for h in pending3:
            h.wait()
        bx, by, bz = bufs[3 * phase], bufs[3 * phase + 1], bufs[3 * phase + 2]

        def p2_iter(it, _, bx=bx, by=by, bz=bz):
            for k in range(U2):
                o = (it * U2 + k) * LANES
                v0 = bx[pl.ds(o, LANES)]
                v1 = by[pl.ds(o, LANES)]
                v2 = bz[pl.ds(o, LANES)]
                i0 = jnp.minimum(((v0 - mn_0) * sc_0).astype(jnp.int32), R - 1)
                i1 = jnp.minimum(((v1 - mn_1) * sc_1).astype(jnp.int32), R - 1)
                i2 = jnp.minimum(((v2 - mn_2) * sc_2).astype(jnp.int32), R - 1)
                vox = (i0 * R + i1) * R + i2 + lane_off
                plsc.addupdate_scatter(hist, [vox], ones16)
            return 0

        lax.fori_loop(0, n2_iters, p2_iter, 0)
        pending3 = nxt3

    inv_n = jnp.float32(1.0 / N)

    def merge_body(g, _):
        acc = zeros16
        for l in range(LANES):
            acc = acc + hist[pl.ds(l * NVOX + g * LANES, LANES)]
        cnt[pl.ds(g * LANES, LANES)] = acc * inv_n
        return 0

    lax.fori_loop(0, NVOX // LANES, merge_body, 0)
    pltpu.sync_copy(cnt, counts_hbm.at[bid])


def _sc_counts(xt):
    _, B, N = xt.shape
    mesh = plsc.VectorSubcoreMesh(core_axis_name="c", subcore_axis_name="s",
                                  num_cores=NC, num_subcores=NS)
    return pl.kernel(
        _hist_body,
        out_type=jax.ShapeDtypeStruct((B, NVOX), jnp.float32),
        mesh=mesh,
        compiler_params=pltpu.CompilerParams(needs_layout_passes=False),
        scratch_types=[
            pltpu.VMEM((CHUNK,), jnp.float32),
            pltpu.VMEM((CHUNK,), jnp.float32),
            pltpu.VMEM((CHUNK,), jnp.float32),
            pltpu.VMEM((CHUNK,), jnp.float32),
            pltpu.VMEM((CHUNK,), jnp.float32),
            pltpu.VMEM((CHUNK,), jnp.float32),
            pltpu.VMEM((LANES * NVOX,), jnp.float32),
            pltpu.VMEM((NVOX,), jnp.float32),
            pltpu.SemaphoreType.DMA,
            pltpu.SemaphoreType.DMA,
            pltpu.SemaphoreType.DMA,
            pltpu.SemaphoreType.DMA,
            pltpu.SemaphoreType.DMA,
            pltpu.SemaphoreType.DMA,
        ],
    )(xt)


def _gemm_body(c_ref, w_ref, b_ref, o_ref):
    o_ref[...] = lax.dot_general(
        c_ref[...], w_ref[...], (((1,), (1,)), ((), ())),
        preferred_element_type=jnp.float32) + b_ref[...]


def _tc_gemm(counts, W, b):
    B = counts.shape[0]
    C = W.shape[0]
    return pl.pallas_call(
        _gemm_body,
        out_shape=jax.ShapeDtypeStruct((B, C), jnp.float32),
    )(counts, W, b.reshape(1, C))


@jax.jit
def kernel(x, W, b):
    xt = jnp.transpose(x, (2, 0, 1))
    counts = _sc_counts(xt)
    return _tc_gemm(counts, W, b)
